# Initial kernel scaffold; baseline (speedup 1.0000x reference)
import jax
import jax.numpy as jnp
from jax.experimental import pallas as pl


def kernel(x, pos, table, alpha):
    raise NotImplementedError("write your pallas kernel here")


# speedup vs baseline: 1.3151x; 1.3151x over previous
import functools

import jax
import jax.numpy as jnp
from jax import lax
from jax.experimental import pallas as pl
from jax.experimental.pallas import tpu as pltpu
from jax.experimental.pallas import tpu_sc as plsc

D = 768
N_ROWS = 4 * 8192
NC, NS, L = 2, 16, 16
NW = NC * NS
ROWS_PER_W = N_ROWS // NW
C = 32
N_CHUNKS = ROWS_PER_W // C
LANES_PER_ROW = D // L


def _sc_body(x_hbm, idx_hbm, table_hbm, alpha_hbm, out_hbm,
             idx_v, rows_v, x_v, alpha_v, sem_g, sem_x):
    wid = lax.axis_index("s") * NC + lax.axis_index("c")
    w_base = wid * ROWS_PER_W

    pltpu.sync_copy(idx_hbm.at[pl.ds(w_base, ROWS_PER_W)], idx_v)
    pltpu.sync_copy(alpha_hbm, alpha_v)
    alpha = alpha_v[...]

    @pl.loop(0, N_CHUNKS)
    def _chunk(i):
        base = w_base + i * C
        gather = pltpu.make_async_copy(
            table_hbm.at[idx_v.at[pl.ds(i * C, C)]], rows_v, sem_g)
        gather.start()
        xcopy = pltpu.make_async_copy(x_hbm.at[pl.ds(base, C), :], x_v, sem_x)
        xcopy.start()
        gather.wait()
        xcopy.wait()

        @pl.loop(0, C)
        def _row(r):
            for j in range(LANES_PER_ROW):
                sl = pl.ds(j * L, L)
                x_v[r, sl] = rows_v[r, sl] * alpha + x_v[r, sl]

        pltpu.sync_copy(x_v, out_hbm.at[pl.ds(base, C), :])


@jax.jit
def _sc_call(x2, idx, table, alpha16):
    mesh = plsc.VectorSubcoreMesh(
        core_axis_name="c", subcore_axis_name="s", num_cores=NC,
        num_subcores=NS)
    return pl.kernel(
        _sc_body,
        out_type=jax.ShapeDtypeStruct((N_ROWS, D), jnp.float32),
        mesh=mesh,
        scratch_types=[
            pltpu.VMEM((ROWS_PER_W,), jnp.int32),
            pltpu.VMEM((C, D), jnp.float32),
            pltpu.VMEM((C, D), jnp.float32),
            pltpu.VMEM((L,), jnp.float32),
            pltpu.SemaphoreType.DMA,
            pltpu.SemaphoreType.DMA,
        ],
    )(x2, idx, table, alpha16)


def kernel(x, pos, table, alpha):
    b, s, d = x.shape
    x2 = x.reshape(b * s, d)
    idx = pos.reshape(b * s)
    alpha16 = jnp.broadcast_to(alpha, (L,))
    out = _sc_call(x2, idx, table, alpha16)
    return out.reshape(b, s, d)
